# Initial kernel scaffold; baseline (speedup 1.0000x reference)
import jax
import jax.numpy as jnp
from jax.experimental import pallas as pl


def kernel(x, weights):
    raise NotImplementedError("write your pallas kernel here")


# speedup vs baseline: 414.4537x; 414.4537x over previous
import jax
import jax.numpy as jnp
from jax import lax
from jax.experimental import pallas as pl
from jax.experimental.pallas import tpu as pltpu

ALPHA_HALF = 0.5
N_STEPS = 4096


def _ksom_body(x0_ref, x1_ref, wc_ref, w_ref, outw_ref, wins_ref):
    w00_0 = wc_ref[0, 0]
    w01_0 = wc_ref[0, 1]
    w10_0 = wc_ref[1, 0]
    w11_0 = wc_ref[1, 1]

    def step(i, c):
        w00, w10, w01, w11 = c
        x0 = x0_ref[i]
        x1 = x1_ref[i]
        e0 = x0 - w00
        e1 = x0 - w10
        d1 = e0 * e0
        d2 = e1 * e1
        is0 = d1 < d2
        wins_ref[i] = jnp.where(is0, jnp.int32(0), jnp.int32(1))
        w00 = jnp.where(is0, w00 + ALPHA_HALF * e0, w00)
        w01 = jnp.where(is0, w01 + ALPHA_HALF * (x1 - w01), w01)
        w10 = jnp.where(is0, w10, w10 + ALPHA_HALF * e1)
        w11 = jnp.where(is0, w11, w11 + ALPHA_HALF * (x1 - w11))
        return (w00, w10, w01, w11)

    w00, w10, w01, w11 = lax.fori_loop(
        0, N_STEPS, step, (w00_0, w10_0, w01_0, w11_0))

    row = lax.broadcasted_iota(jnp.int32, (2, 1024), 0)
    col = lax.broadcasted_iota(jnp.int32, (2, 1024), 1)
    corner = jnp.where(row == 0,
                       jnp.where(col == 0, w00, w01),
                       jnp.where(col == 0, w10, w11))
    outw_ref[...] = jnp.where(col < 2, corner, w_ref[...])


def kernel(x, weights):
    x0 = x[:, 0]
    x1 = x[:, 1]
    wcorner = weights[:, :2]
    final_w, wins = pl.pallas_call(
        _ksom_body,
        out_shape=(
            jax.ShapeDtypeStruct((2, 1024), jnp.float32),
            jax.ShapeDtypeStruct((N_STEPS,), jnp.int32),
        ),
        in_specs=[
            pl.BlockSpec(memory_space=pltpu.SMEM),
            pl.BlockSpec(memory_space=pltpu.SMEM),
            pl.BlockSpec(memory_space=pltpu.SMEM),
            pl.BlockSpec(memory_space=pltpu.VMEM),
        ],
        out_specs=(
            pl.BlockSpec(memory_space=pltpu.VMEM),
            pl.BlockSpec(memory_space=pltpu.SMEM),
        ),
    )(x0, x1, wcorner, weights)
    return final_w, wins


# speedup vs baseline: 495.3453x; 1.1952x over previous
import functools

import jax
import jax.numpy as jnp
from jax import lax
from jax.experimental import pallas as pl
from jax.experimental.pallas import tpu as pltpu
from jax.experimental.pallas import tpu_sc as plsc

ALPHA_HALF = 0.5
N_STEPS = 4096
W_FLAT = 2048

_mesh = plsc.VectorSubcoreMesh(core_axis_name="c", subcore_axis_name="s")


@functools.partial(
    pl.kernel,
    out_type=(
        jax.ShapeDtypeStruct((W_FLAT,), jnp.float32),
        jax.ShapeDtypeStruct((N_STEPS,), jnp.int32),
    ),
    mesh=_mesh,
    compiler_params=pltpu.CompilerParams(needs_layout_passes=False),
    scratch_types=[
        pltpu.VMEM((N_STEPS,), jnp.float32),
        pltpu.VMEM((N_STEPS,), jnp.float32),
        pltpu.VMEM((N_STEPS,), jnp.int32),
        pltpu.VMEM((W_FLAT,), jnp.float32),
    ],
)
def _ksom_sc(x0_hbm, x1_hbm, w_hbm, outw_hbm, wins_hbm,
             x0_v, x1_v, wins_v, w_v):
    wid = lax.axis_index("s") * 2 + lax.axis_index("c")

    @pl.when(wid == 0)
    def _():
        pltpu.sync_copy(x0_hbm, x0_v)
        pltpu.sync_copy(x1_hbm, x1_v)
        pltpu.sync_copy(w_hbm, w_v)

        lane = lax.iota(jnp.int32, 16)
        lane0 = lane == 0

        w00_0 = plsc.load_gather(w_v, [jnp.full((16,), 0, jnp.int32)])
        w01_0 = plsc.load_gather(w_v, [jnp.full((16,), 1, jnp.int32)])
        w10_0 = plsc.load_gather(w_v, [jnp.full((16,), 1024, jnp.int32)])
        w11_0 = plsc.load_gather(w_v, [jnp.full((16,), 1025, jnp.int32)])

        def step(i, c):
            w00, w10, w01, w11 = c
            idx = jnp.full((16,), i, jnp.int32)
            x0 = plsc.load_gather(x0_v, [idx])
            x1 = plsc.load_gather(x1_v, [idx])
            e0 = x0 - w00
            e1 = x0 - w10
            is0 = (e0 * e0) < (e1 * e1)
            win = jnp.where(is0, jnp.int32(0), jnp.int32(1))
            plsc.store_scatter(wins_v, [idx], win, mask=lane0)
            w00 = jnp.where(is0, w00 + ALPHA_HALF * e0, w00)
            w01 = jnp.where(is0, w01 + ALPHA_HALF * (x1 - w01), w01)
            w10 = jnp.where(is0, w10, w10 + ALPHA_HALF * e1)
            w11 = jnp.where(is0, w11, w11 + ALPHA_HALF * (x1 - w11))
            return (w00, w10, w01, w11)

        w00, w10, w01, w11 = lax.fori_loop(
            0, N_STEPS, step, (w00_0, w10_0, w01_0, w11_0))

        cidx = jnp.where(lane == 0, 0,
                         jnp.where(lane == 1, 1,
                                   jnp.where(lane == 2, 1024, 1025)))
        vals = jnp.where(lane == 0, w00,
                         jnp.where(lane == 1, w01,
                                   jnp.where(lane == 2, w10, w11)))
        plsc.store_scatter(w_v, [cidx], vals, mask=lane < 4)
        pltpu.sync_copy(w_v, outw_hbm)
        pltpu.sync_copy(wins_v, wins_hbm)


def kernel(x, weights):
    final_w_flat, wins = _ksom_sc(x[:, 0], x[:, 1], weights.reshape(W_FLAT))
    return final_w_flat.reshape(2, 1024), wins
